# baseline (speedup 1.0000x reference)
import jax
import jax.numpy as jnp
from jax import lax
from jax.experimental import pallas as pl
from jax.experimental.pallas import tpu as pltpu
from jax.experimental.pallas import tpu_sc as plsc

N = 100000
NSP = 10
LANES = 16
NC = 2
NS = 16
NW = NC * NS

M_TC = 16384
R_SC = N - M_TC
CHUNK = ((R_SC + NW - 1) // NW + LANES - 1) // LANES * LANES
LAST = R_SC - CHUNK


def _sc_body(ae_hbm, idx_hbm, scale_hbm, shift_hbm, out_hbm,
             ae_v, idx_v, out_v, scale_v, shift_v, sem):
    wid = lax.axis_index("s") * NC + lax.axis_index("c")
    base = jnp.minimum(CHUNK * wid, LAST)
    c1 = pltpu.make_async_copy(idx_hbm.at[pl.ds(base, CHUNK)], idx_v, sem)
    c2 = pltpu.make_async_copy(ae_hbm.at[pl.ds(base, CHUNK)], ae_v, sem)
    c3 = pltpu.make_async_copy(scale_hbm, scale_v.at[pl.ds(0, NSP)], sem)
    c4 = pltpu.make_async_copy(shift_hbm, shift_v.at[pl.ds(0, NSP)], sem)
    c1.start(); c2.start(); c3.start(); c4.start()
    c1.wait(); c2.wait(); c3.wait(); c4.wait()

    @plsc.parallel_loop(0, CHUNK, step=LANES, unroll=4)
    def _(off):
        s = pl.ds(off, LANES)
        iv = idx_v[s]
        av = ae_v[s]
        sc = plsc.load_gather(scale_v, [iv])
        sh = plsc.load_gather(shift_v, [iv])
        out_v[s] = av * sc + sh

    pltpu.sync_copy(out_v, out_hbm.at[pl.ds(base, CHUNK)])


_sc_call = pl.kernel(
    _sc_body,
    out_type=jax.ShapeDtypeStruct((R_SC,), jnp.float32),
    mesh=plsc.VectorSubcoreMesh(
        core_axis_name="c", subcore_axis_name="s",
        num_cores=NC, num_subcores=NS),
    compiler_params=pltpu.CompilerParams(needs_layout_passes=False),
    scratch_types=[
        pltpu.VMEM((CHUNK,), jnp.float32),
        pltpu.VMEM((CHUNK,), jnp.int32),
        pltpu.VMEM((CHUNK,), jnp.float32),
        pltpu.VMEM((LANES,), jnp.float32),
        pltpu.VMEM((LANES,), jnp.float32),
        pltpu.SemaphoreType.DMA,
    ],
)


def _tc_body(ae_ref, idx_ref, scale_ref, shift_ref, out_ref):
    ae = ae_ref[...]
    idx = idx_ref[...]
    sc = jnp.full(ae.shape, scale_ref[0], jnp.float32)
    sh = jnp.full(ae.shape, shift_ref[0], jnp.float32)
    for s in range(1, NSP):
        m = idx == s
        sc = jnp.where(m, scale_ref[s], sc)
        sh = jnp.where(m, shift_ref[s], sh)
    out_ref[...] = ae * sc + sh


_tc_call = pl.pallas_call(
    _tc_body,
    out_shape=jax.ShapeDtypeStruct((M_TC // 128, 128), jnp.float32),
    in_specs=[
        pl.BlockSpec(memory_space=pltpu.VMEM),
        pl.BlockSpec(memory_space=pltpu.VMEM),
        pl.BlockSpec(memory_space=pltpu.SMEM),
        pl.BlockSpec(memory_space=pltpu.SMEM),
    ],
    out_specs=pl.BlockSpec(memory_space=pltpu.VMEM),
)


def kernel(atomic_energy, atom_number, scale, shift):
    ae = atomic_energy.reshape(-1).astype(jnp.float32)
    idx = atom_number.reshape(-1).astype(jnp.int32)
    scale = scale.astype(jnp.float32)
    shift = shift.astype(jnp.float32)
    tc_out = _tc_call(ae[:M_TC].reshape(M_TC // 128, 128),
                      idx[:M_TC].reshape(M_TC // 128, 128),
                      scale, shift).reshape(-1)
    sc_out = _sc_call(ae[M_TC:], idx[M_TC:], scale, shift)
    return jnp.concatenate([tc_out, sc_out])


# speedup vs baseline: 1.1163x; 1.1163x over previous
import jax
import jax.numpy as jnp
from jax import lax
from jax.experimental import pallas as pl
from jax.experimental.pallas import tpu as pltpu
from jax.experimental.pallas import tpu_sc as plsc

N = 100000
NC = 2
NS = 16
NW = NC * NS
LANES = 16
CHUNK = 3136
LAST = N - CHUNK
NSP = 10


def _sc_body(ae_hbm, idx_hbm, scale_hbm, shift_hbm, out_hbm,
             ae_v, idx_v, out_v, scale_v, shift_v, sem):
    wid = lax.axis_index("s") * NC + lax.axis_index("c")
    base = jnp.minimum(CHUNK * wid, LAST)
    c1 = pltpu.make_async_copy(idx_hbm.at[pl.ds(base, CHUNK)], idx_v, sem)
    c2 = pltpu.make_async_copy(ae_hbm.at[pl.ds(base, CHUNK)], ae_v, sem)
    c3 = pltpu.make_async_copy(scale_hbm, scale_v.at[pl.ds(0, NSP)], sem)
    c4 = pltpu.make_async_copy(shift_hbm, shift_v.at[pl.ds(0, NSP)], sem)
    c1.start(); c2.start(); c3.start(); c4.start()
    c1.wait(); c2.wait(); c3.wait(); c4.wait()

    @plsc.parallel_loop(0, CHUNK, step=LANES, unroll=8)
    def _(off):
        s = pl.ds(off, LANES)
        iv = idx_v[s]
        av = ae_v[s]
        sc = plsc.load_gather(scale_v, [iv])
        sh = plsc.load_gather(shift_v, [iv])
        out_v[s] = av * sc + sh

    pltpu.sync_copy(out_v, out_hbm.at[pl.ds(base, CHUNK)])


_sc_call = pl.kernel(
    _sc_body,
    out_type=jax.ShapeDtypeStruct((N,), jnp.float32),
    mesh=plsc.VectorSubcoreMesh(
        core_axis_name="c", subcore_axis_name="s",
        num_cores=NC, num_subcores=NS),
    compiler_params=pltpu.CompilerParams(needs_layout_passes=False),
    scratch_types=[
        pltpu.VMEM((CHUNK,), jnp.float32),
        pltpu.VMEM((CHUNK,), jnp.int32),
        pltpu.VMEM((CHUNK,), jnp.float32),
        pltpu.VMEM((LANES,), jnp.float32),
        pltpu.VMEM((LANES,), jnp.float32),
        pltpu.SemaphoreType.DMA,
    ],
)


def kernel(atomic_energy, atom_number, scale, shift):
    ae = atomic_energy.reshape(-1).astype(jnp.float32)
    idx = atom_number.reshape(-1).astype(jnp.int32)
    return _sc_call(ae, idx, scale.astype(jnp.float32),
                    shift.astype(jnp.float32))
